# baseline (speedup 1.0000x reference)
import functools

import jax
import jax.numpy as jnp
from jax import lax
from jax.experimental import pallas as pl
from jax.experimental.pallas import tpu as pltpu
from jax.experimental.pallas import tpu_sc as plsc

L = 16
CHUNK = 128
NBUF = 2
TC_BLK = 4000

_info = plsc.get_sparse_core_info()
NW = _info.num_cores * _info.num_subcores


def _tc_copy(cell, hidden):
    N, D = cell.shape
    nb = N // TC_BLK

    def body(c_ref, h_ref, o_ref):
        o_ref[0] = c_ref[...]
        o_ref[1] = h_ref[...]

    return pl.pallas_call(
        body,
        grid=(nb,),
        in_specs=[
            pl.BlockSpec((TC_BLK, D), lambda i: (i, 0)),
            pl.BlockSpec((TC_BLK, D), lambda i: (i, 0)),
        ],
        out_specs=pl.BlockSpec((2, TC_BLK, D), lambda i: (0, i, 0)),
        out_shape=jax.ShapeDtypeStruct((2, N, D), jnp.float32),
    )(cell, hidden)


def _sc_update(out_ref, idx, values_cell, values_hidden, n_rows):
    N = n_rows
    B = idx.shape[0]
    D = values_cell.shape[1]
    R8 = -(-N // NW // 8) * 8
    rpad = ((R8 + L - 1) // L) * L
    wcap = (rpad // CHUNK + 2) * CHUNK

    @functools.partial(
        pl.kernel,
        mesh=plsc.VectorSubcoreMesh(core_axis_name="c", subcore_axis_name="s"),
        compiler_params=pltpu.CompilerParams(needs_layout_passes=False),
        scratch_types=[
            pltpu.VMEM((B,), jnp.int32),
            pltpu.VMEM((rpad,), jnp.int32),
            pltpu.VMEM((wcap,), jnp.int32),
            pltpu.VMEM((wcap,), jnp.int32),
            pltpu.VMEM((2, NBUF, CHUNK), jnp.int32),
            pltpu.VMEM((NBUF, CHUNK), jnp.int32),
            pltpu.VMEM((2, NBUF, CHUNK, D), jnp.float32),
        ] + [pltpu.SemaphoreType.DMA] * (4 * NBUF),
    )
    def k(idx_h, vc_h, vh_h, out_h,
          idx_v, tmp, win_row, win_pos, dst2d, src2d, stage, *sems):
        sems_g = (sems[:NBUF], sems[NBUF:2 * NBUF])
        sems_s = (sems[2 * NBUF:3 * NBUF], sems[3 * NBUF:])
        wid = lax.axis_index("s") * _info.num_cores + lax.axis_index("c")
        lo = wid * R8
        hi = jnp.minimum(lo + R8, N)

        pltpu.sync_copy(idx_h, idx_v)

        neg1 = jnp.full((L,), -1, jnp.int32)

        def init_body(i, _):
            tmp[pl.ds(i * L, L)] = neg1
            return 0
        lax.fori_loop(0, rpad // L, init_body, 0)

        iota = lax.iota(jnp.int32, L)

        def p1(v, _):
            for u in range(4):
                rows = idx_v[pl.ds((v * 4 + u) * L, L)]
                m = (rows >= lo) & (rows < hi)
                local = jnp.where(m, rows - lo, 0)
                pos = iota + (v * 4 + u) * L
                _, last_m = plsc.scan_count(local, mask=m)
                plsc.store_scatter(tmp, [local], pos, mask=last_m & m)
            return 0
        lax.fori_loop(0, B // L // 4, p1, 0)

        def p2(t, cnt):
            w = tmp[pl.ds(t * L, L)]
            m = w >= 0
            rows16 = iota + t * L + lo
            plsc.store_compressed(win_row.at[pl.ds(cnt, L)], rows16, mask=m)
            plsc.store_compressed(win_pos.at[pl.ds(cnt, L)], w, mask=m)
            return cnt + jnp.sum(m.astype(jnp.int32))
        cnt = lax.fori_loop(0, rpad // L, p2, jnp.int32(0))

        @pl.when(cnt > 0)
        def _pad():
            frv = jnp.full((L,), win_row[pl.ds(0, L)][0], jnp.int32)
            fpv = jnp.full((L,), win_pos[pl.ds(0, L)][0], jnp.int32)
            for j in range(CHUNK // L):
                win_row[pl.ds(cnt + j * L, L)] = frv
                win_pos[pl.ds(cnt + j * L, L)] = fpv

        nch_w = (cnt + CHUNK - 1) // CHUNK

        def drain(sem, t, b):
            pltpu.make_async_copy(
                vc_h.at[pl.ds(0, CHUNK)], stage.at[t, b], sem).wait()

        def do_group(g, _):
            for b in range(NBUF):
                c = g * NBUF + b

                @pl.when(c < nch_w)
                def _(b=b, c=c):
                    def ld(j, _):
                        s2 = src2d.at[b]
                        d2c = dst2d.at[0, b]
                        d2h = dst2d.at[1, b]
                        wr = win_row[pl.ds(c * CHUNK + j * L, L)]
                        d2c[pl.ds(j * L, L)] = wr
                        d2h[pl.ds(j * L, L)] = wr + N
                        s2[pl.ds(j * L, L)] = win_pos[
                            pl.ds(c * CHUNK + j * L, L)]
                        return 0
                    lax.fori_loop(0, CHUNK // L, ld, 0)
                    pltpu.make_async_copy(
                        vc_h.at[src2d.at[b]], stage.at[0, b],
                        sems_g[0][b]).start()
                    pltpu.make_async_copy(
                        vh_h.at[src2d.at[b]], stage.at[1, b],
                        sems_g[1][b]).start()
            for b in range(NBUF):
                c = g * NBUF + b

                @pl.when(c < nch_w)
                def _(b=b, c=c):
                    for t in range(2):
                        drain(sems_g[t][b], t, b)
                        pltpu.make_async_copy(
                            stage.at[t, b], out_h.at[dst2d.at[t, b]],
                            sems_s[t][b]).start()
            for b in range(NBUF):
                c = g * NBUF + b

                @pl.when(c < nch_w)
                def _(b=b, c=c):
                    for t in range(2):
                        drain(sems_s[t][b], t, b)
            return 0

        ngroups = (nch_w + NBUF - 1) // NBUF
        lax.fori_loop(0, ngroups, do_group, 0)

    k(idx, values_cell, values_hidden, out_ref)


def kernel(cell, hidden, node_idxs, values_cell, values_hidden):
    N, D = cell.shape
    idx = node_idxs.astype(jnp.int32)
    out0 = _tc_copy(cell, hidden)
    out_ref = jax.new_ref(out0.reshape(2 * N, D))
    _sc_update(out_ref, idx, values_cell, values_hidden, N)
    return out_ref[...].reshape(2, N, D)


# speedup vs baseline: 1.1056x; 1.1056x over previous
import functools

import jax
import jax.numpy as jnp
from jax import lax
from jax.experimental import pallas as pl
from jax.experimental.pallas import tpu as pltpu
from jax.experimental.pallas import tpu_sc as plsc

L = 16
CHUNK = 128
NB = 6
TC_BLK = 4000

_info = plsc.get_sparse_core_info()
NW = _info.num_cores * _info.num_subcores
_MESH = dict(core_axis_name="c", subcore_axis_name="s")


def _worker_id():
    return lax.axis_index("s") * _info.num_cores + lax.axis_index("c")


def _sc_scan(idx, n_rows):
    B = idx.shape[0]
    N = n_rows
    R8 = -(-N // NW // 8) * 8
    rpad = ((R8 + L - 1) // L) * L
    wcap = (rpad // CHUNK + 2) * CHUNK

    @functools.partial(
        pl.kernel,
        out_type=(
            jax.ShapeDtypeStruct((NW, wcap), jnp.int32),
            jax.ShapeDtypeStruct((NW, wcap), jnp.int32),
            jax.ShapeDtypeStruct((NW, L), jnp.int32),
        ),
        mesh=plsc.VectorSubcoreMesh(**_MESH),
        compiler_params=pltpu.CompilerParams(needs_layout_passes=False),
        scratch_types=[
            pltpu.VMEM((B,), jnp.int32),
            pltpu.VMEM((rpad,), jnp.int32),
            pltpu.VMEM((wcap,), jnp.int32),
            pltpu.VMEM((wcap,), jnp.int32),
            pltpu.VMEM((L,), jnp.int32),
        ],
    )
    def k(idx_h, wrow_h, wpos_h, nch_h, idx_v, tmp, win_row, win_pos, nch_v):
        wid = _worker_id()
        lo = wid * R8
        hi = jnp.minimum(lo + R8, N)

        pltpu.sync_copy(idx_h, idx_v)

        neg1 = jnp.full((L,), -1, jnp.int32)

        def init_body(i, _):
            tmp[pl.ds(i * L, L)] = neg1
            return 0
        lax.fori_loop(0, rpad // L, init_body, 0)

        iota = lax.iota(jnp.int32, L)

        def p1(v, _):
            rows = idx_v[pl.ds(v * L, L)]
            m = (rows >= lo) & (rows < hi)
            local = jnp.where(m, rows - lo, 0)
            pos = iota + v * L
            _, last_m = plsc.scan_count(local, mask=m)
            plsc.store_scatter(tmp, [local], pos, mask=last_m & m)
            return 0
        lax.fori_loop(0, B // L, p1, 0)

        def p2(t, cnt):
            w = tmp[pl.ds(t * L, L)]
            m = w >= 0
            rows16 = iota + t * L + lo
            plsc.store_compressed(win_row.at[pl.ds(cnt, L)], rows16, mask=m)
            plsc.store_compressed(win_pos.at[pl.ds(cnt, L)], w, mask=m)
            return cnt + plsc.all_reduce_population_count(m)[0]
        cnt = lax.fori_loop(0, rpad // L, p2, jnp.int32(0))

        @pl.when(cnt > 0)
        def _pad():
            frv = jnp.full((L,), win_row[pl.ds(0, L)][0], jnp.int32)
            fpv = jnp.full((L,), win_pos[pl.ds(0, L)][0], jnp.int32)
            for j in range(CHUNK // L):
                win_row[pl.ds(cnt + j * L, L)] = frv
                win_pos[pl.ds(cnt + j * L, L)] = fpv

        nch_v[pl.ds(0, L)] = jnp.full((L,), (cnt + CHUNK - 1) // CHUNK,
                                      jnp.int32)
        pltpu.sync_copy(win_row, wrow_h.at[wid])
        pltpu.sync_copy(win_pos, wpos_h.at[wid])
        pltpu.sync_copy(nch_v, nch_h.at[wid])

    return k(idx)


def _tc_copy(cell, hidden):
    N, D = cell.shape
    nb = N // TC_BLK

    def body(c_ref, h_ref, o_ref):
        o_ref[0] = c_ref[...]
        o_ref[1] = h_ref[...]

    return pl.pallas_call(
        body,
        grid=(nb,),
        in_specs=[
            pl.BlockSpec((TC_BLK, D), lambda i: (i, 0)),
            pl.BlockSpec((TC_BLK, D), lambda i: (i, 0)),
        ],
        out_specs=pl.BlockSpec((2, TC_BLK, D), lambda i: (0, i, 0)),
        out_shape=jax.ShapeDtypeStruct((2, N, D), jnp.float32),
    )(cell, hidden)


def _sc_scatter(out_ref, wrow, wpos, nch, values_cell, values_hidden, n_rows):
    N = n_rows
    D = values_cell.shape[1]
    wcap = wrow.shape[1]

    @functools.partial(
        pl.kernel,
        mesh=plsc.VectorSubcoreMesh(**_MESH),
        compiler_params=pltpu.CompilerParams(needs_layout_passes=False),
        scratch_types=[
            pltpu.VMEM((wcap,), jnp.int32),
            pltpu.VMEM((wcap,), jnp.int32),
            pltpu.VMEM((L,), jnp.int32),
            pltpu.VMEM((NB, CHUNK), jnp.int32),
            pltpu.VMEM((NB, CHUNK), jnp.int32),
            pltpu.VMEM((NB, CHUNK, D), jnp.float32),
        ] + [pltpu.SemaphoreType.DMA] * (2 * NB),
    )
    def k(wrow_h, wpos_h, nch_h, vc_h, vh_h, out_h,
          win_row, win_pos, nch_v, dst2d, src2d, stage, *sems):
        sems_g = sems[:NB]
        sems_s = sems[NB:]
        wid = _worker_id()
        pltpu.sync_copy(wrow_h.at[wid], win_row)
        pltpu.sync_copy(wpos_h.at[wid], win_pos)
        pltpu.sync_copy(nch_h.at[wid], nch_v)
        nch_w = nch_v[pl.ds(0, L)][0]
        ntot = 2 * nch_w

        def fire_gather(b, c):
            t_is_h = c >= nch_w
            ci = jnp.where(t_is_h, c - nch_w, c)
            base = jnp.where(t_is_h, N, 0)

            def ld(j, _):
                d2 = dst2d.at[b]
                s2 = src2d.at[b]
                d2[pl.ds(j * L, L)] = (
                    win_row[pl.ds(ci * CHUNK + j * L, L)] + base)
                s2[pl.ds(j * L, L)] = win_pos[pl.ds(ci * CHUNK + j * L, L)]
                return 0
            lax.fori_loop(0, CHUNK // L, ld, 0)

            @pl.when(jnp.logical_not(t_is_h))
            def _():
                pltpu.make_async_copy(
                    vc_h.at[src2d.at[b]], stage.at[b], sems_g[b]).start()

            @pl.when(t_is_h)
            def _():
                pltpu.make_async_copy(
                    vh_h.at[src2d.at[b]], stage.at[b], sems_g[b]).start()

        def drain(sem, b):
            pltpu.make_async_copy(
                vc_h.at[pl.ds(0, CHUNK)], stage.at[b], sem).wait()

        for b in range(NB):
            @pl.when(b < ntot)
            def _(b=b):
                fire_gather(b, jnp.int32(b))

        def ring(g, _):
            for b in range(NB):
                c = g * NB + b

                @pl.when(c < ntot)
                def _(b=b, c=c):
                    drain(sems_g[b], b)
                    pltpu.make_async_copy(
                        stage.at[b], out_h.at[dst2d.at[b]], sems_s[b]).start()
            for b in range(NB):
                c2 = (g + 1) * NB + b

                @pl.when(c2 < ntot)
                def _(b=b, c2=c2):
                    drain(sems_s[b], b)
                    fire_gather(b, c2)
            return 0
        ngroups = (ntot + NB - 1) // NB
        lax.fori_loop(0, ngroups, ring, 0)
        for b in range(NB):
            @pl.when(b < ntot)
            def _(b=b):
                drain(sems_s[b], b)

    k(wrow, wpos, nch, values_cell, values_hidden, out_ref)


def kernel(cell, hidden, node_idxs, values_cell, values_hidden):
    N, D = cell.shape
    idx = node_idxs.astype(jnp.int32)
    out0 = _tc_copy(cell, hidden)
    wrow, wpos, nch = _sc_scan(idx, N)
    out_ref = jax.new_ref(out0.reshape(2 * N, D))
    _sc_scatter(out_ref, wrow, wpos, nch, values_cell, values_hidden, N)
    return out_ref[...].reshape(2, N, D)
